# baseline (speedup 1.0000x reference)
import jax
import jax.numpy as jnp
from jax.experimental import pallas as pl
from jax.experimental.pallas import tpu as pltpu

_T, _D, _E, _F = 2048, 768, 8, 3072
_K = 512
_FB = 768
_NF = _F // _FB


def _router(x, gate_w):
    scores = jnp.dot(x, gate_w, preferred_element_type=jnp.float32)
    probs = jax.nn.softmax(scores, axis=-1)
    pbits = jax.lax.bitcast_convert_type(probs, jnp.int32)

    def vstep(_, carry):
        lo, hi = carry
        mid = (lo + hi) // 2
        cnt = jnp.sum((pbits >= mid).astype(jnp.int32), axis=0, keepdims=True)
        big = cnt >= _K
        return jnp.where(big, mid, lo), jnp.where(big, hi, mid)

    lo0 = jnp.zeros((1, _E), jnp.int32)
    hi0 = jnp.full((1, _E), 0x7F800000, jnp.int32)
    v, _ = jax.lax.fori_loop(0, 31, vstep, (lo0, hi0))

    gt = pbits > v
    eq = pbits == v
    idx = jax.lax.broadcasted_iota(jnp.int32, (_T, _E), 0)

    def tstep(_, carry):
        lo, hi = carry
        mid = (lo + hi) // 2
        cnt = jnp.sum((gt | (eq & (idx < mid))).astype(jnp.int32),
                      axis=0, keepdims=True)
        big = cnt >= _K
        return jnp.where(big, lo, mid), jnp.where(big, mid, hi)

    tlo0 = jnp.zeros((1, _E), jnp.int32)
    thi0 = jnp.full((1, _E), _T, jnp.int32)
    _, tthr = jax.lax.fori_loop(0, 11, tstep, (tlo0, thi0))

    sel = gt | (eq & (idx < tthr))
    return jnp.where(sel, probs, 0.0)


def _moe_kernel(x_ref, gw_ref, w1_ref, w2_ref, out_ref, gsel_ref, acc_ref):
    e = pl.program_id(0)
    fi = pl.program_id(1)
    step = e * _NF + fi

    @pl.when(step == 0)
    def _():
        gsel_ref[...] = _router(x_ref[0], gw_ref[...])

    x16 = x_ref[0].astype(jnp.bfloat16)
    w1 = w1_ref[0].astype(jnp.bfloat16)
    w2 = w2_ref[0].astype(jnp.bfloat16)
    h = jnp.dot(x16, w1, preferred_element_type=jnp.float32)
    h = jax.nn.gelu(h).astype(jnp.bfloat16)
    h2 = jnp.dot(h, w2, preferred_element_type=jnp.float32)

    lane = jax.lax.broadcasted_iota(jnp.int32, (_T, _E), 1)
    g = jnp.sum(jnp.where(lane == e, gsel_ref[...], 0.0), axis=1, keepdims=True)
    contrib = g * h2

    @pl.when(step == 0)
    def _():
        acc_ref[...] = contrib

    @pl.when(step > 0)
    def _():
        acc_ref[...] += contrib

    @pl.when(step == _E * _NF - 1)
    def _():
        out_ref[0] = acc_ref[...].astype(jnp.bfloat16)


def kernel(x, w1, w2, gate_w):
    x = x.astype(jnp.float32)
    out = pl.pallas_call(
        _moe_kernel,
        grid=(_E, _NF),
        in_specs=[
            pl.BlockSpec((1, _T, _D), lambda e, f: (0, 0, 0)),
            pl.BlockSpec((_D, _E), lambda e, f: (0, 0)),
            pl.BlockSpec((1, _D, _FB), lambda e, f: (e, 0, f)),
            pl.BlockSpec((1, _FB, _D), lambda e, f: (e, f, 0)),
        ],
        out_specs=pl.BlockSpec((1, _T, _D), lambda e, f: (0, 0, 0)),
        out_shape=jax.ShapeDtypeStruct((1, _T, _D), jnp.bfloat16),
        scratch_shapes=[
            pltpu.VMEM((_T, _E), jnp.float32),
            pltpu.VMEM((_T, _D), jnp.float32),
        ],
        compiler_params=pltpu.CompilerParams(
            dimension_semantics=("arbitrary", "arbitrary"),
        ),
    )(x, gate_w, w1, w2)
    return out


# speedup vs baseline: 1.9388x; 1.9388x over previous
import jax
import jax.numpy as jnp
from jax.experimental import pallas as pl
from jax.experimental.pallas import tpu as pltpu

_T, _D, _E, _F = 2048, 768, 8, 3072
_K = 512
_FB = 768
_NF = _F // _FB


def _tdot(a, b):
    return jax.lax.dot_general(a, b, (((0,), (0,)), ((), ())),
                               preferred_element_type=jnp.float32)


def _router(x, gate_w):
    scores = jnp.dot(x, gate_w, preferred_element_type=jnp.float32)
    probs = jax.nn.softmax(scores, axis=-1)
    pbits = jax.lax.bitcast_convert_type(probs, jnp.int32)

    def vstep(_, carry):
        lo, hi = carry
        mid = (lo + hi) // 2
        cnt = jnp.sum((pbits >= mid).astype(jnp.int32), axis=0, keepdims=True)
        big = cnt >= _K
        return jnp.where(big, mid, lo), jnp.where(big, hi, mid)

    lo0 = jnp.zeros((1, _E), jnp.int32)
    hi0 = jnp.full((1, _E), 0x7F800000, jnp.int32)
    v, _ = jax.lax.fori_loop(0, 31, vstep, (lo0, hi0))

    gt = pbits > v
    eq = pbits == v
    idx = jax.lax.broadcasted_iota(jnp.int32, (_T, _E), 0)

    def tstep(_, carry):
        lo, hi = carry
        mid = (lo + hi) // 2
        cnt = jnp.sum((gt | (eq & (idx < mid))).astype(jnp.int32),
                      axis=0, keepdims=True)
        big = cnt >= _K
        return jnp.where(big, lo, mid), jnp.where(big, mid, hi)

    tlo0 = jnp.zeros((1, _E), jnp.int32)
    thi0 = jnp.full((1, _E), _T, jnp.int32)
    _, tthr = jax.lax.fori_loop(0, 11, tstep, (tlo0, thi0))

    sel = gt | (eq & (idx < tthr))
    gsel = jnp.where(sel, probs, 0.0)

    r = jax.lax.broadcasted_iota(jnp.int32, (_T, _T), 0)
    c = jax.lax.broadcasted_iota(jnp.int32, (_T, _T), 1)
    lt = (r > c).astype(jnp.bfloat16)
    rank = jnp.dot(lt, sel.astype(jnp.bfloat16),
                   preferred_element_type=jnp.float32)
    return gsel, rank


def _moe_kernel(x_ref, gw_ref, w1_ref, w2_ref, out_ref,
                gsel_ref, rank_ref, ptg_ref, xc_ref, accc_ref, acc_ref):
    e = pl.program_id(0)
    fi = pl.program_id(1)
    step = e * _NF + fi

    @pl.when(step == 0)
    def _():
        gsel, rank = _router(x_ref[0], gw_ref[...])
        gsel_ref[...] = gsel
        rank_ref[...] = rank
        acc_ref[...] = jnp.zeros((_T, _D), jnp.float32)

    @pl.when(fi == 0)
    def _():
        lane = jax.lax.broadcasted_iota(jnp.int32, (_T, _E), 1)
        emask = lane == e
        g_col = jnp.sum(jnp.where(emask, gsel_ref[...], 0.0),
                        axis=1, keepdims=True)
        r_col = jnp.sum(jnp.where(emask, rank_ref[...], 0.0),
                        axis=1, keepdims=True)
        sel_col = g_col > 0.0
        kio = jax.lax.broadcasted_iota(jnp.int32, (_T, _K), 1)
        r_int = r_col.astype(jnp.int32)
        pt = jnp.where((kio == r_int) & sel_col, 1.0, 0.0)
        ptg_ref[...] = (pt * g_col).astype(jnp.bfloat16)
        xc = _tdot(pt.astype(jnp.bfloat16),
                   x_ref[0].astype(jnp.bfloat16))
        xc_ref[...] = xc.astype(jnp.bfloat16)
        accc_ref[...] = jnp.zeros((_K, _D), jnp.float32)

    w1 = w1_ref[0].astype(jnp.bfloat16)
    w2 = w2_ref[0].astype(jnp.bfloat16)
    h = jnp.dot(xc_ref[...], w1, preferred_element_type=jnp.float32)
    h = jax.nn.gelu(h).astype(jnp.bfloat16)
    accc_ref[...] += jnp.dot(h, w2, preferred_element_type=jnp.float32)

    @pl.when(fi == _NF - 1)
    def _():
        acc_ref[...] += jnp.dot(ptg_ref[...], accc_ref[...].astype(jnp.bfloat16),
                                preferred_element_type=jnp.float32)

    @pl.when(step == _E * _NF - 1)
    def _():
        out_ref[0] = acc_ref[...].astype(jnp.bfloat16)


def kernel(x, w1, w2, gate_w):
    x = x.astype(jnp.float32)
    out = pl.pallas_call(
        _moe_kernel,
        grid=(_E, _NF),
        in_specs=[
            pl.BlockSpec((1, _T, _D), lambda e, f: (0, 0, 0)),
            pl.BlockSpec((_D, _E), lambda e, f: (0, 0)),
            pl.BlockSpec((1, _D, _FB), lambda e, f: (e, 0, f)),
            pl.BlockSpec((1, _FB, _D), lambda e, f: (e, f, 0)),
        ],
        out_specs=pl.BlockSpec((1, _T, _D), lambda e, f: (0, 0, 0)),
        out_shape=jax.ShapeDtypeStruct((1, _T, _D), jnp.bfloat16),
        scratch_shapes=[
            pltpu.VMEM((_T, _E), jnp.float32),
            pltpu.VMEM((_T, _E), jnp.float32),
            pltpu.VMEM((_T, _K), jnp.bfloat16),
            pltpu.VMEM((_K, _D), jnp.bfloat16),
            pltpu.VMEM((_K, _D), jnp.float32),
            pltpu.VMEM((_T, _D), jnp.float32),
        ],
        compiler_params=pltpu.CompilerParams(
            dimension_semantics=("arbitrary", "arbitrary"),
        ),
    )(x, gate_w, w1, w2)
    return out


# speedup vs baseline: 2.1829x; 1.1259x over previous
import jax
import jax.numpy as jnp
from jax.experimental import pallas as pl
from jax.experimental.pallas import tpu as pltpu

_T, _D, _E, _F = 2048, 768, 8, 3072
_K = 512
_FB = 768
_NF = _F // _FB


def _tdot(a, b):
    return jax.lax.dot_general(a, b, (((0,), (0,)), ((), ())),
                               preferred_element_type=jnp.float32)


def _router(x, gate_w):
    scores = jnp.dot(x, gate_w, preferred_element_type=jnp.float32)
    probs = jax.nn.softmax(scores, axis=-1)
    pbits = jax.lax.bitcast_convert_type(probs, jnp.int32)

    def vstep(_, carry):
        lo, hi = carry
        mid = (lo + hi) // 2
        cnt = jnp.sum((pbits >= mid).astype(jnp.int32), axis=0, keepdims=True)
        big = cnt >= _K
        return jnp.where(big, mid, lo), jnp.where(big, hi, mid)

    lo0 = jnp.zeros((1, _E), jnp.int32)
    hi0 = jnp.full((1, _E), 0x7F800000, jnp.int32)
    v, _ = jax.lax.fori_loop(0, 31, vstep, (lo0, hi0))

    gt = pbits > v
    eq = pbits == v
    idx = jax.lax.broadcasted_iota(jnp.int32, (_T, _E), 0)

    def tstep(_, carry):
        lo, hi = carry
        mid = (lo + hi) // 2
        cnt = jnp.sum((gt | (eq & (idx < mid))).astype(jnp.int32),
                      axis=0, keepdims=True)
        big = cnt >= _K
        return jnp.where(big, lo, mid), jnp.where(big, mid, hi)

    tlo0 = jnp.zeros((1, _E), jnp.int32)
    thi0 = jnp.full((1, _E), _T, jnp.int32)
    _, tthr = jax.lax.fori_loop(0, 11, tstep, (tlo0, thi0))

    sel = gt | (eq & (idx < tthr))
    gsel = jnp.where(sel, probs, 0.0)

    rank = sel.astype(jnp.float32)
    s = 1
    while s < _T:
        rank = rank + jnp.concatenate(
            [jnp.zeros((s, _E), jnp.float32), rank[:-s]], axis=0)
        s *= 2
    rank = rank - sel.astype(jnp.float32)
    return gsel, rank


def _moe_kernel(x_ref, gw_ref, w1_ref, w2_ref, out_ref,
                gsel_ref, rank_ref, acc_ref):
    e = pl.program_id(0)

    @pl.when(e == 0)
    def _():
        gsel, rank = _router(x_ref[0], gw_ref[...])
        gsel_ref[...] = gsel
        rank_ref[...] = rank
        acc_ref[...] = jnp.zeros((_T, _D), jnp.float32)

    lane = jax.lax.broadcasted_iota(jnp.int32, (_T, _E), 1)
    emask = lane == e
    g_col = jnp.sum(jnp.where(emask, gsel_ref[...], 0.0),
                    axis=1, keepdims=True)
    r_col = jnp.sum(jnp.where(emask, rank_ref[...], 0.0),
                    axis=1, keepdims=True)
    sel_col = g_col > 0.0
    kio = jax.lax.broadcasted_iota(jnp.int32, (_T, _K), 1)
    r_int = r_col.astype(jnp.int32)
    pt32 = jnp.where((kio == r_int) & sel_col, 1.0, 0.0)
    pt = pt32.astype(jnp.bfloat16)
    ptg = (pt32 * g_col).astype(jnp.bfloat16)
    xc = _tdot(pt, x_ref[0].astype(jnp.bfloat16)).astype(jnp.bfloat16)

    w1 = w1_ref[0].astype(jnp.bfloat16)
    w2 = w2_ref[0].astype(jnp.bfloat16)
    h = jnp.dot(xc, w1, preferred_element_type=jnp.float32)
    h = jax.nn.gelu(h).astype(jnp.bfloat16)
    h2 = jnp.dot(h, w2, preferred_element_type=jnp.float32)
    acc_ref[...] += jnp.dot(ptg, h2.astype(jnp.bfloat16),
                            preferred_element_type=jnp.float32)

    @pl.when(e == _E - 1)
    def _():
        out_ref[0] = acc_ref[...].astype(jnp.bfloat16)


def kernel(x, w1, w2, gate_w):
    x = x.astype(jnp.float32)
    out = pl.pallas_call(
        _moe_kernel,
        grid=(_E,),
        in_specs=[
            pl.BlockSpec((1, _T, _D), lambda e: (0, 0, 0)),
            pl.BlockSpec((_D, _E), lambda e: (0, 0)),
            pl.BlockSpec((1, _D, _F), lambda e: (e, 0, 0)),
            pl.BlockSpec((1, _F, _D), lambda e: (e, 0, 0)),
        ],
        out_specs=pl.BlockSpec((1, _T, _D), lambda e: (0, 0, 0)),
        out_shape=jax.ShapeDtypeStruct((1, _T, _D), jnp.bfloat16),
        scratch_shapes=[
            pltpu.VMEM((_T, _E), jnp.float32),
            pltpu.VMEM((_T, _E), jnp.float32),
            pltpu.VMEM((_T, _D), jnp.float32),
        ],
        compiler_params=pltpu.CompilerParams(
            dimension_semantics=("arbitrary",),
            vmem_limit_bytes=110 * 1024 * 1024,
        ),
    )(x, gate_w, w1, w2)
    return out
